# baseline (speedup 1.0000x reference)
import functools

import jax
import jax.numpy as jnp
from jax import lax
from jax.experimental import pallas as pl
from jax.experimental.pallas import tpu as pltpu
from jax.experimental.pallas import tpu_sc as plsc

B = 16384
D = 64
LANES = 16
PACK = 2
PW = PACK * D
NC = 2
NS = 16
NW = NC * NS
BPW = B // NW
CH = 256
NCH = BPW // CH
IDC = 128
GPC = CH // LANES


def _body(uid_hbm, iid_hbm, pu_hbm, pi_hbm, out_hbm,
          uidv, iidv, ugidx, igidx, ubuf, ibuf, outv, sem):
    wid = lax.axis_index("s") * NC + lax.axis_index("c")
    base = wid * BPW

    pltpu.sync_copy(uid_hbm.at[pl.ds(base, BPW)], uidv)
    pltpu.sync_copy(iid_hbm.at[pl.ds(base, BPW)], iidv)

    for g in range(BPW // LANES):
        r, c = g // (IDC // LANES), (g % (IDC // LANES)) * LANES
        ugidx[r, pl.ds(c, LANES)] = jax.lax.shift_right_logical(
            uidv[pl.ds(g * LANES, LANES)], 1)
        igidx[r, pl.ds(c, LANES)] = jax.lax.shift_right_logical(
            iidv[pl.ds(g * LANES, LANES)], 1)

    lane = lax.iota(jnp.int32, LANES)

    for ch in range(NCH):
        copies = []
        for r in range(CH // IDC):
            row = ch * (CH // IDC) + r
            copies.append(pltpu.async_copy(
                pu_hbm.at[ugidx.at[row]], ubuf.at[pl.ds(r * IDC, IDC)], sem))
            copies.append(pltpu.async_copy(
                pi_hbm.at[igidx.at[row]], ibuf.at[pl.ds(r * IDC, IDC)], sem))
        for c in copies:
            c.wait()

        def group(g, carry):
            uids16 = uidv[pl.ds(ch * CH + g * LANES, LANES)]
            iids16 = iidv[pl.ds(ch * CH + g * LANES, LANES)]
            ucol = (uids16 & 1) * D
            icol = (iids16 & 1) * D
            rows = g * LANES + lane
            acc = jnp.zeros((LANES,), jnp.float32)
            for d in range(D):
                uu = plsc.load_gather(ubuf, [rows, ucol + d])
                vv = plsc.load_gather(ibuf, [rows, icol + d])
                acc = acc + uu * vv
            outv[pl.ds(ch * CH + g * LANES, LANES)] = acc
            return carry

        lax.fori_loop(0, GPC, group, 0)

    pltpu.sync_copy(outv, out_hbm.at[pl.ds(base, BPW)])


def kernel(user_ids, item_ids, user_table, item_table):
    nrows = user_table.shape[0] // PACK
    pu = user_table.reshape(nrows, PW)
    pi = item_table.reshape(nrows, PW)
    mesh = plsc.VectorSubcoreMesh(core_axis_name="c", subcore_axis_name="s")
    run = functools.partial(
        pl.kernel,
        mesh=mesh,
        compiler_params=pltpu.CompilerParams(needs_layout_passes=False),
        out_type=jax.ShapeDtypeStruct((B,), jnp.float32),
        scratch_types=[
            pltpu.VMEM((BPW,), jnp.int32),
            pltpu.VMEM((BPW,), jnp.int32),
            pltpu.VMEM((BPW // IDC, IDC), jnp.int32),
            pltpu.VMEM((BPW // IDC, IDC), jnp.int32),
            pltpu.VMEM((CH, PW), jnp.float32),
            pltpu.VMEM((CH, PW), jnp.float32),
            pltpu.VMEM((BPW,), jnp.float32),
            pltpu.SemaphoreType.DMA,
        ],
    )(_body)
    return run(user_ids.astype(jnp.int32), item_ids.astype(jnp.int32), pu, pi)


# speedup vs baseline: 1.1657x; 1.1657x over previous
import functools

import jax
import jax.numpy as jnp
from jax import lax
from jax.experimental import pallas as pl
from jax.experimental.pallas import tpu as pltpu
from jax.experimental.pallas import tpu_sc as plsc

B = 16384
D = 64
LANES = 16
PACK = 2
PW = PACK * D
NC = 2
NS = 16
NW = NC * NS
BPW = B // NW
CH = 256
NCH = BPW // CH
IDC = 128
GPC = CH // LANES

RBLK = 1024


def _repack_body(ut_ref, it_ref, pu_ref, pi_ref):
    z = jnp.zeros((RBLK, PW - D), jnp.float32)
    pu_ref[:, 0:D] = jnp.transpose(ut_ref[...], (1, 0))
    pu_ref[:, D:PW] = z
    pi_ref[:, 0:D] = jnp.transpose(it_ref[...], (1, 0))
    pi_ref[:, D:PW] = z


def _sc_body(uid_hbm, iid_hbm, pu_hbm, pi_hbm, out_hbm,
             uidv, iidv, ugidx, igidx, ubuf, ibuf, outv, sem):
    wid = lax.axis_index("s") * NC + lax.axis_index("c")
    base = wid * BPW

    pltpu.sync_copy(uid_hbm.at[pl.ds(base, BPW)], uidv)
    pltpu.sync_copy(iid_hbm.at[pl.ds(base, BPW)], iidv)

    for g in range(BPW // LANES):
        r, c = g // (IDC // LANES), (g % (IDC // LANES)) * LANES
        ugidx[r, pl.ds(c, LANES)] = uidv[pl.ds(g * LANES, LANES)]
        igidx[r, pl.ds(c, LANES)] = iidv[pl.ds(g * LANES, LANES)]

    lane = lax.iota(jnp.int32, LANES)

    for ch in range(NCH):
        copies = []
        for r in range(CH // IDC):
            row = ch * (CH // IDC) + r
            copies.append(pltpu.async_copy(
                pu_hbm.at[ugidx.at[row]], ubuf.at[pl.ds(r * IDC, IDC)], sem))
            copies.append(pltpu.async_copy(
                pi_hbm.at[igidx.at[row]], ibuf.at[pl.ds(r * IDC, IDC)], sem))
        for c in copies:
            c.wait()

        def group(g, carry):
            rows = g * LANES + lane
            acc = jnp.zeros((LANES,), jnp.float32)
            for d in range(D):
                col = jnp.full((LANES,), d, jnp.int32)
                uu = plsc.load_gather(ubuf, [rows, col])
                vv = plsc.load_gather(ibuf, [rows, col])
                acc = acc + uu * vv
            outv[pl.ds(ch * CH + g * LANES, LANES)] = acc
            return carry

        lax.fori_loop(0, GPC, group, 0)

    pltpu.sync_copy(outv, out_hbm.at[pl.ds(base, BPW)])


def kernel(user_ids, item_ids, user_table, item_table):
    n = user_table.shape[0]
    utT = user_table.T
    itT = item_table.T
    grid = pl.cdiv(n, RBLK)

    pu, pi = pl.pallas_call(
        _repack_body,
        grid=(grid,),
        in_specs=[
            pl.BlockSpec((D, RBLK), lambda g: (0, g)),
            pl.BlockSpec((D, RBLK), lambda g: (0, g)),
        ],
        out_specs=[
            pl.BlockSpec((RBLK, PW), lambda g: (g, 0)),
            pl.BlockSpec((RBLK, PW), lambda g: (g, 0)),
        ],
        out_shape=[
            jax.ShapeDtypeStruct((n, PW), jnp.float32),
            jax.ShapeDtypeStruct((n, PW), jnp.float32),
        ],
    )(utT, itT)

    mesh = plsc.VectorSubcoreMesh(core_axis_name="c", subcore_axis_name="s")
    run = functools.partial(
        pl.kernel,
        mesh=mesh,
        compiler_params=pltpu.CompilerParams(needs_layout_passes=False),
        out_type=jax.ShapeDtypeStruct((B,), jnp.float32),
        scratch_types=[
            pltpu.VMEM((BPW,), jnp.int32),
            pltpu.VMEM((BPW,), jnp.int32),
            pltpu.VMEM((BPW // IDC, IDC), jnp.int32),
            pltpu.VMEM((BPW // IDC, IDC), jnp.int32),
            pltpu.VMEM((CH, PW), jnp.float32),
            pltpu.VMEM((CH, PW), jnp.float32),
            pltpu.VMEM((BPW,), jnp.float32),
            pltpu.SemaphoreType.DMA,
        ],
    )(_sc_body)
    return run(user_ids.astype(jnp.int32), item_ids.astype(jnp.int32), pu, pi)


# speedup vs baseline: 1.6290x; 1.3974x over previous
import functools

import jax
import jax.numpy as jnp
from jax import lax
from jax.experimental import pallas as pl
from jax.experimental.pallas import tpu as pltpu
from jax.experimental.pallas import tpu_sc as plsc

B = 16384
D = 64
LANES = 16
PACK = 2
PW = PACK * D
NC = 2
NS = 16
NW = NC * NS
BPW = B // NW
CH = 256
NCH = BPW // CH
IDC = 128
GPC = CH // LANES

RBLK = 2048


def _repack_body(ut_ref, it_ref, pc_ref):
    pc_ref[...] = jnp.concatenate(
        [jnp.transpose(ut_ref[...], (1, 0)),
         jnp.transpose(it_ref[...], (1, 0))], axis=1)


def _sc_body(uid_hbm, iid_hbm, pc_hbm, out_hbm,
             uidv, iidv, ugidx, igidx, ubuf, ibuf, outv, sem):
    wid = lax.axis_index("s") * NC + lax.axis_index("c")
    base = wid * BPW

    pltpu.sync_copy(uid_hbm.at[pl.ds(base, BPW)], uidv)
    pltpu.sync_copy(iid_hbm.at[pl.ds(base, BPW)], iidv)

    for g in range(BPW // LANES):
        r, c = g // (IDC // LANES), (g % (IDC // LANES)) * LANES
        ugidx[r, pl.ds(c, LANES)] = uidv[pl.ds(g * LANES, LANES)]
        igidx[r, pl.ds(c, LANES)] = iidv[pl.ds(g * LANES, LANES)]

    lane = lax.iota(jnp.int32, LANES)

    for ch in range(NCH):
        copies = []
        for r in range(CH // IDC):
            row = ch * (CH // IDC) + r
            copies.append(pltpu.async_copy(
                pc_hbm.at[ugidx.at[row]], ubuf.at[pl.ds(r * IDC, IDC)], sem))
            copies.append(pltpu.async_copy(
                pc_hbm.at[igidx.at[row]], ibuf.at[pl.ds(r * IDC, IDC)], sem))
        for c in copies:
            c.wait()

        def group(g, carry):
            rows = g * LANES + lane
            acc = jnp.zeros((LANES,), jnp.float32)
            for d in range(D):
                ucol = jnp.full((LANES,), d, jnp.int32)
                icol = jnp.full((LANES,), D + d, jnp.int32)
                uu = plsc.load_gather(ubuf, [rows, ucol])
                vv = plsc.load_gather(ibuf, [rows, icol])
                acc = acc + uu * vv
            outv[pl.ds(ch * CH + g * LANES, LANES)] = acc
            return carry

        lax.fori_loop(0, GPC, group, 0)

    pltpu.sync_copy(outv, out_hbm.at[pl.ds(base, BPW)])


def kernel(user_ids, item_ids, user_table, item_table):
    n = user_table.shape[0]
    utT = user_table.T
    itT = item_table.T
    grid = pl.cdiv(n, RBLK)

    pc = pl.pallas_call(
        _repack_body,
        grid=(grid,),
        in_specs=[
            pl.BlockSpec((D, RBLK), lambda g: (0, g)),
            pl.BlockSpec((D, RBLK), lambda g: (0, g)),
        ],
        out_specs=pl.BlockSpec((RBLK, PW), lambda g: (g, 0)),
        out_shape=jax.ShapeDtypeStruct((n, PW), jnp.float32),
    )(utT, itT)

    mesh = plsc.VectorSubcoreMesh(core_axis_name="c", subcore_axis_name="s")
    run = functools.partial(
        pl.kernel,
        mesh=mesh,
        compiler_params=pltpu.CompilerParams(needs_layout_passes=False),
        out_type=jax.ShapeDtypeStruct((B,), jnp.float32),
        scratch_types=[
            pltpu.VMEM((BPW,), jnp.int32),
            pltpu.VMEM((BPW,), jnp.int32),
            pltpu.VMEM((BPW // IDC, IDC), jnp.int32),
            pltpu.VMEM((BPW // IDC, IDC), jnp.int32),
            pltpu.VMEM((CH, PW), jnp.float32),
            pltpu.VMEM((CH, PW), jnp.float32),
            pltpu.VMEM((BPW,), jnp.float32),
            pltpu.SemaphoreType.DMA,
        ],
    )(_sc_body)
    return run(user_ids.astype(jnp.int32), item_ids.astype(jnp.int32), pc)


# speedup vs baseline: 2.0246x; 1.2429x over previous
import functools

import jax
import jax.numpy as jnp
from jax import lax
from jax.experimental import pallas as pl
from jax.experimental.pallas import tpu as pltpu
from jax.experimental.pallas import tpu_sc as plsc

B = 16384
D = 64
LANES = 16
PACK = 2
PW = PACK * D
NC = 2
NS = 16
NW = NC * NS
BPW = B // NW
CH = 256
NCH = BPW // CH
IDC = 128
GPC = CH // LANES

RBLK = 4096


def _repack_body(ut_ref, it_ref, pc_ref):
    pc_ref[...] = jnp.concatenate(
        [jnp.transpose(ut_ref[...], (1, 0)),
         jnp.transpose(it_ref[...], (1, 0))], axis=1)


def _sc_body(uid_hbm, iid_hbm, pc_hbm, out_hbm,
             uidv, iidv, ugidx, igidx, ubuf, ibuf, outv, sem):
    wid = lax.axis_index("s") * NC + lax.axis_index("c")
    base = wid * BPW

    pltpu.sync_copy(uid_hbm.at[pl.ds(base, BPW)], uidv)
    pltpu.sync_copy(iid_hbm.at[pl.ds(base, BPW)], iidv)

    for g in range(BPW // LANES):
        r, c = g // (IDC // LANES), (g % (IDC // LANES)) * LANES
        ugidx[r, pl.ds(c, LANES)] = uidv[pl.ds(g * LANES, LANES)]
        igidx[r, pl.ds(c, LANES)] = iidv[pl.ds(g * LANES, LANES)]

    lane = lax.iota(jnp.int32, LANES)

    for ch in range(NCH):
        copies = []
        for r in range(CH // IDC):
            row = ch * (CH // IDC) + r
            copies.append(pltpu.async_copy(
                pc_hbm.at[ugidx.at[row]], ubuf.at[pl.ds(r * IDC, IDC)], sem))
            copies.append(pltpu.async_copy(
                pc_hbm.at[igidx.at[row]], ibuf.at[pl.ds(r * IDC, IDC)], sem))
        for c in copies:
            c.wait()

        def group(g, carry):
            rows = g * LANES + lane
            acc = jnp.zeros((LANES,), jnp.float32)
            for d in range(D):
                ucol = jnp.full((LANES,), d, jnp.int32)
                icol = jnp.full((LANES,), D + d, jnp.int32)
                uu = plsc.load_gather(ubuf, [rows, ucol])
                vv = plsc.load_gather(ibuf, [rows, icol])
                acc = acc + uu * vv
            outv[pl.ds(ch * CH + g * LANES, LANES)] = acc
            return carry

        lax.fori_loop(0, GPC, group, 0)

    pltpu.sync_copy(outv, out_hbm.at[pl.ds(base, BPW)])


def kernel(user_ids, item_ids, user_table, item_table):
    n = user_table.shape[0]
    utT = user_table.T
    itT = item_table.T
    grid = pl.cdiv(n, RBLK)

    pc = pl.pallas_call(
        _repack_body,
        grid=(grid,),
        in_specs=[
            pl.BlockSpec((D, RBLK), lambda g: (0, g)),
            pl.BlockSpec((D, RBLK), lambda g: (0, g)),
        ],
        out_specs=pl.BlockSpec((RBLK, PW), lambda g: (g, 0)),
        out_shape=jax.ShapeDtypeStruct((n, PW), jnp.float32),
    )(utT, itT)

    mesh = plsc.VectorSubcoreMesh(core_axis_name="c", subcore_axis_name="s")
    run = functools.partial(
        pl.kernel,
        mesh=mesh,
        compiler_params=pltpu.CompilerParams(needs_layout_passes=False),
        out_type=jax.ShapeDtypeStruct((B,), jnp.float32),
        scratch_types=[
            pltpu.VMEM((BPW,), jnp.int32),
            pltpu.VMEM((BPW,), jnp.int32),
            pltpu.VMEM((BPW // IDC, IDC), jnp.int32),
            pltpu.VMEM((BPW // IDC, IDC), jnp.int32),
            pltpu.VMEM((CH, PW), jnp.float32),
            pltpu.VMEM((CH, PW), jnp.float32),
            pltpu.VMEM((BPW,), jnp.float32),
            pltpu.SemaphoreType.DMA,
        ],
    )(_sc_body)
    return run(user_ids.astype(jnp.int32), item_ids.astype(jnp.int32), pc)


# speedup vs baseline: 2.2993x; 1.1357x over previous
import functools

import jax
import jax.numpy as jnp
from jax import lax
from jax.experimental import pallas as pl
from jax.experimental.pallas import tpu as pltpu
from jax.experimental.pallas import tpu_sc as plsc

B = 16384
D = 64
LANES = 16
PACK = 2
PW = PACK * D
NC = 2
NS = 16
NW = NC * NS
BPW = B // NW
CH = 256
NCH = BPW // CH
IDC = 128
GPC = CH // LANES

RBLK = 8192


def _repack_body(ut_ref, it_ref, pc_ref):
    pc_ref[...] = jnp.concatenate(
        [jnp.transpose(ut_ref[...], (1, 0)),
         jnp.transpose(it_ref[...], (1, 0))], axis=1)


def _sc_body(uid_hbm, iid_hbm, pc_hbm, out_hbm,
             uidv, iidv, ugidx, igidx, ubuf, ibuf, outv, sem):
    wid = lax.axis_index("s") * NC + lax.axis_index("c")
    base = wid * BPW

    pltpu.sync_copy(uid_hbm.at[pl.ds(base, BPW)], uidv)
    pltpu.sync_copy(iid_hbm.at[pl.ds(base, BPW)], iidv)

    for g in range(BPW // LANES):
        r, c = g // (IDC // LANES), (g % (IDC // LANES)) * LANES
        ugidx[r, pl.ds(c, LANES)] = uidv[pl.ds(g * LANES, LANES)]
        igidx[r, pl.ds(c, LANES)] = iidv[pl.ds(g * LANES, LANES)]

    lane = lax.iota(jnp.int32, LANES)

    for ch in range(NCH):
        copies = []
        for r in range(CH // IDC):
            row = ch * (CH // IDC) + r
            copies.append(pltpu.async_copy(
                pc_hbm.at[ugidx.at[row]], ubuf.at[pl.ds(r * IDC, IDC)], sem))
            copies.append(pltpu.async_copy(
                pc_hbm.at[igidx.at[row]], ibuf.at[pl.ds(r * IDC, IDC)], sem))
        for c in copies:
            c.wait()

        def group(g, carry):
            rows = g * LANES + lane
            acc = jnp.zeros((LANES,), jnp.float32)
            for d in range(D):
                ucol = jnp.full((LANES,), d, jnp.int32)
                icol = jnp.full((LANES,), D + d, jnp.int32)
                uu = plsc.load_gather(ubuf, [rows, ucol])
                vv = plsc.load_gather(ibuf, [rows, icol])
                acc = acc + uu * vv
            outv[pl.ds(ch * CH + g * LANES, LANES)] = acc
            return carry

        lax.fori_loop(0, GPC, group, 0)

    pltpu.sync_copy(outv, out_hbm.at[pl.ds(base, BPW)])


def kernel(user_ids, item_ids, user_table, item_table):
    n = user_table.shape[0]
    utT = user_table.T
    itT = item_table.T
    grid = pl.cdiv(n, RBLK)

    pc = pl.pallas_call(
        _repack_body,
        grid=(grid,),
        in_specs=[
            pl.BlockSpec((D, RBLK), lambda g: (0, g)),
            pl.BlockSpec((D, RBLK), lambda g: (0, g)),
        ],
        out_specs=pl.BlockSpec((RBLK, PW), lambda g: (g, 0)),
        out_shape=jax.ShapeDtypeStruct((n, PW), jnp.float32),
    )(utT, itT)

    mesh = plsc.VectorSubcoreMesh(core_axis_name="c", subcore_axis_name="s")
    run = functools.partial(
        pl.kernel,
        mesh=mesh,
        compiler_params=pltpu.CompilerParams(needs_layout_passes=False),
        out_type=jax.ShapeDtypeStruct((B,), jnp.float32),
        scratch_types=[
            pltpu.VMEM((BPW,), jnp.int32),
            pltpu.VMEM((BPW,), jnp.int32),
            pltpu.VMEM((BPW // IDC, IDC), jnp.int32),
            pltpu.VMEM((BPW // IDC, IDC), jnp.int32),
            pltpu.VMEM((CH, PW), jnp.float32),
            pltpu.VMEM((CH, PW), jnp.float32),
            pltpu.VMEM((BPW,), jnp.float32),
            pltpu.SemaphoreType.DMA,
        ],
    )(_sc_body)
    return run(user_ids.astype(jnp.int32), item_ids.astype(jnp.int32), pc)


# speedup vs baseline: 2.4486x; 1.0649x over previous
import functools

import jax
import jax.numpy as jnp
from jax import lax
from jax.experimental import pallas as pl
from jax.experimental.pallas import tpu as pltpu
from jax.experimental.pallas import tpu_sc as plsc

B = 16384
D = 64
LANES = 16
PACK = 2
PW = PACK * D
NC = 2
NS = 16
NW = NC * NS
BPW = B // NW
CH = 256
NCH = BPW // CH
IDC = 128
GPC = CH // LANES

RBLK = 16384


def _repack_body(ut_ref, it_ref, pc_ref):
    pc_ref[...] = jnp.concatenate(
        [jnp.transpose(ut_ref[...], (1, 0)),
         jnp.transpose(it_ref[...], (1, 0))], axis=1)


def _sc_body(uid_hbm, iid_hbm, pc_hbm, out_hbm,
             uidv, iidv, ugidx, igidx, ubuf, ibuf, outv, sem):
    wid = lax.axis_index("s") * NC + lax.axis_index("c")
    base = wid * BPW

    pltpu.sync_copy(uid_hbm.at[pl.ds(base, BPW)], uidv)
    pltpu.sync_copy(iid_hbm.at[pl.ds(base, BPW)], iidv)

    for g in range(BPW // LANES):
        r, c = g // (IDC // LANES), (g % (IDC // LANES)) * LANES
        ugidx[r, pl.ds(c, LANES)] = uidv[pl.ds(g * LANES, LANES)]
        igidx[r, pl.ds(c, LANES)] = iidv[pl.ds(g * LANES, LANES)]

    lane = lax.iota(jnp.int32, LANES)

    for ch in range(NCH):
        copies = []
        for r in range(CH // IDC):
            row = ch * (CH // IDC) + r
            copies.append(pltpu.async_copy(
                pc_hbm.at[ugidx.at[row]], ubuf.at[pl.ds(r * IDC, IDC)], sem))
            copies.append(pltpu.async_copy(
                pc_hbm.at[igidx.at[row]], ibuf.at[pl.ds(r * IDC, IDC)], sem))
        for c in copies:
            c.wait()

        def group(g, carry):
            rows = g * LANES + lane
            acc = jnp.zeros((LANES,), jnp.float32)
            for d in range(D):
                ucol = jnp.full((LANES,), d, jnp.int32)
                icol = jnp.full((LANES,), D + d, jnp.int32)
                uu = plsc.load_gather(ubuf, [rows, ucol])
                vv = plsc.load_gather(ibuf, [rows, icol])
                acc = acc + uu * vv
            outv[pl.ds(ch * CH + g * LANES, LANES)] = acc
            return carry

        lax.fori_loop(0, GPC, group, 0)

    pltpu.sync_copy(outv, out_hbm.at[pl.ds(base, BPW)])


def kernel(user_ids, item_ids, user_table, item_table):
    n = user_table.shape[0]
    utT = user_table.T
    itT = item_table.T
    grid = pl.cdiv(n, RBLK)

    pc = pl.pallas_call(
        _repack_body,
        grid=(grid,),
        in_specs=[
            pl.BlockSpec((D, RBLK), lambda g: (0, g)),
            pl.BlockSpec((D, RBLK), lambda g: (0, g)),
        ],
        out_specs=pl.BlockSpec((RBLK, PW), lambda g: (g, 0)),
        out_shape=jax.ShapeDtypeStruct((n, PW), jnp.float32),
    )(utT, itT)

    mesh = plsc.VectorSubcoreMesh(core_axis_name="c", subcore_axis_name="s")
    run = functools.partial(
        pl.kernel,
        mesh=mesh,
        compiler_params=pltpu.CompilerParams(needs_layout_passes=False),
        out_type=jax.ShapeDtypeStruct((B,), jnp.float32),
        scratch_types=[
            pltpu.VMEM((BPW,), jnp.int32),
            pltpu.VMEM((BPW,), jnp.int32),
            pltpu.VMEM((BPW // IDC, IDC), jnp.int32),
            pltpu.VMEM((BPW // IDC, IDC), jnp.int32),
            pltpu.VMEM((CH, PW), jnp.float32),
            pltpu.VMEM((CH, PW), jnp.float32),
            pltpu.VMEM((BPW,), jnp.float32),
            pltpu.SemaphoreType.DMA,
        ],
    )(_sc_body)
    return run(user_ids.astype(jnp.int32), item_ids.astype(jnp.int32), pc)


# speedup vs baseline: 2.4683x; 1.0081x over previous
import functools

import jax
import jax.numpy as jnp
from jax import lax
from jax.experimental import pallas as pl
from jax.experimental.pallas import tpu as pltpu
from jax.experimental.pallas import tpu_sc as plsc

B = 16384
D = 64
LANES = 16
PACK = 2
PW = PACK * D
NC = 2
NS = 16
NW = NC * NS
BPW = B // NW
CH = 128
NCH = BPW // CH
IDC = 128
GPC = CH // LANES

RBLK = 16384


def _repack_body(ut_ref, it_ref, pc_ref):
    pc_ref[...] = jnp.concatenate(
        [jnp.transpose(ut_ref[...], (1, 0)),
         jnp.transpose(it_ref[...], (1, 0))], axis=1)


def _sc_body(uid_hbm, iid_hbm, pc_hbm, out_hbm,
             uidv, iidv, ugidx, igidx,
             ubuf0, ibuf0, ubuf1, ibuf1, outv, sem0, sem1):
    wid = lax.axis_index("s") * NC + lax.axis_index("c")
    base = wid * BPW

    pltpu.sync_copy(uid_hbm.at[pl.ds(base, BPW)], uidv)
    pltpu.sync_copy(iid_hbm.at[pl.ds(base, BPW)], iidv)

    for g in range(BPW // LANES):
        r, c = g // (IDC // LANES), (g % (IDC // LANES)) * LANES
        ugidx[r, pl.ds(c, LANES)] = uidv[pl.ds(g * LANES, LANES)]
        igidx[r, pl.ds(c, LANES)] = iidv[pl.ds(g * LANES, LANES)]

    lane = lax.iota(jnp.int32, LANES)
    bufs = [(ubuf0, ibuf0, sem0), (ubuf1, ibuf1, sem1)]

    def fire(ch):
        ub, ib, sem = bufs[ch % 2]
        return (pltpu.async_copy(pc_hbm.at[ugidx.at[ch]], ub, sem),
                pltpu.async_copy(pc_hbm.at[igidx.at[ch]], ib, sem))

    inflight = fire(0)
    for ch in range(NCH):
        for cp in inflight:
            cp.wait()
        if ch + 1 < NCH:
            inflight = fire(ch + 1)
        ub, ib, _ = bufs[ch % 2]

        def group(g, carry):
            rows = g * LANES + lane
            acc = jnp.zeros((LANES,), jnp.float32)
            for d in range(D):
                ucol = jnp.full((LANES,), d, jnp.int32)
                icol = jnp.full((LANES,), D + d, jnp.int32)
                uu = plsc.load_gather(ub, [rows, ucol])
                vv = plsc.load_gather(ib, [rows, icol])
                acc = acc + uu * vv
            outv[pl.ds(ch * CH + g * LANES, LANES)] = acc
            return carry

        lax.fori_loop(0, GPC, group, 0)

    pltpu.sync_copy(outv, out_hbm.at[pl.ds(base, BPW)])


def kernel(user_ids, item_ids, user_table, item_table):
    n = user_table.shape[0]
    utT = user_table.T
    itT = item_table.T
    grid = pl.cdiv(n, RBLK)

    pc = pl.pallas_call(
        _repack_body,
        grid=(grid,),
        in_specs=[
            pl.BlockSpec((D, RBLK), lambda g: (0, g)),
            pl.BlockSpec((D, RBLK), lambda g: (0, g)),
        ],
        out_specs=pl.BlockSpec((RBLK, PW), lambda g: (g, 0)),
        out_shape=jax.ShapeDtypeStruct((n, PW), jnp.float32),
    )(utT, itT)

    mesh = plsc.VectorSubcoreMesh(core_axis_name="c", subcore_axis_name="s")
    run = functools.partial(
        pl.kernel,
        mesh=mesh,
        compiler_params=pltpu.CompilerParams(needs_layout_passes=False),
        out_type=jax.ShapeDtypeStruct((B,), jnp.float32),
        scratch_types=[
            pltpu.VMEM((BPW,), jnp.int32),
            pltpu.VMEM((BPW,), jnp.int32),
            pltpu.VMEM((BPW // IDC, IDC), jnp.int32),
            pltpu.VMEM((BPW // IDC, IDC), jnp.int32),
            pltpu.VMEM((CH, PW), jnp.float32),
            pltpu.VMEM((CH, PW), jnp.float32),
            pltpu.VMEM((CH, PW), jnp.float32),
            pltpu.VMEM((CH, PW), jnp.float32),
            pltpu.VMEM((BPW,), jnp.float32),
            pltpu.SemaphoreType.DMA,
            pltpu.SemaphoreType.DMA,
        ],
    )(_sc_body)
    return run(user_ids.astype(jnp.int32), item_ids.astype(jnp.int32), pc)
